# baseline (speedup 1.0000x reference)
import functools

import jax
import jax.numpy as jnp
from jax.experimental import pallas as pl
from jax.experimental.pallas import tpu as pltpu


_NV = 64
_P = 8
_BIN_CHUNK = 4


def _band_hist_kernel(st_ref, out_ref, sc_ref, *, nbins, nb, rows):
    g = pl.program_id(1)
    for chunk in range(0, nbins, _BIN_CHUNK):

        accs = [
            jnp.zeros((rows, out_ref.shape[-1]), jnp.int16)
            for _ in range(_BIN_CHUNK)
        ]
        for b in range(nb):
            if chunk == 0:
                tile = st_ref[0, b, 0, :, :].astype(jnp.int16)
                sc_ref[b, :, :] = tile
            else:
                tile = sc_ref[b, :, :]
            for i in range(_BIN_CHUNK):
                accs[i] = accs[i] + (
                    tile == jnp.int16(chunk + i)
                ).astype(jnp.int16)
        for i, acc in enumerate(accs):
            red = (acc[0:8, :] + acc[8:16, :]) + (acc[16:24, :] + acc[24:32, :])

            @pl.when(g == 0)
            def _(red=red, i=i, chunk=chunk):
                out_ref[0, chunk + i, :, :] = red

            @pl.when(g != 0)
            def _(red=red, i=i, chunk=chunk):
                out_ref[0, chunk + i, :, :] += red


def _band_counts(seg, nV, P):
    B, H, W = seg.shape
    rows = H // P

    ngroup = 2
    nb = B // ngroup
    st = seg.reshape(ngroup, nb, P, rows, W)

    kernel_body = functools.partial(
        _band_hist_kernel, nbins=nV, nb=nb, rows=rows
    )

    return pl.pallas_call(
        kernel_body,
        out_shape=jax.ShapeDtypeStruct((P, nV, 8, W), jnp.int16),
        grid=(P, ngroup),
        in_specs=[
            pl.BlockSpec((1, nb, 1, rows, W), lambda hp, g: (g, 0, hp, 0, 0))
        ],
        out_specs=pl.BlockSpec((1, nV, 8, W), lambda hp, g: (hp, 0, 0, 0)),
        scratch_shapes=[pltpu.VMEM((nb, rows, W), jnp.int16)],
        compiler_params=pltpu.CompilerParams(
            dimension_semantics=("parallel", "arbitrary")
        ),
    )(st)


def kernel(seg, byx):
    del byx
    nV, P = _NV, _P
    pps = P
    B, H, W = seg.shape
    ws = W // P

    partial = _band_counts(seg.astype(jnp.int32), nV, P)

    counts = partial.reshape(P, nV, 8, P, ws).sum(
        axis=(2, 4), dtype=jnp.int32
    )
    grid = (
        counts.transpose(1, 0, 2)
        .astype(jnp.float32)
        .reshape(nV, 1, P, P)
    )
    sizes = counts.sum(axis=(0, 2)).astype(jnp.float32)
    den = sizes * (pps / 32.0) ** 2
    return grid / den.reshape(-1, 1, 1, 1)


# speedup vs baseline: 1.2099x; 1.2099x over previous
import functools

import jax
import jax.numpy as jnp
from jax.experimental import pallas as pl
from jax.experimental.pallas import tpu as pltpu


_NV = 64
_P = 8
_BIN_CHUNK = 8


def _band_hist_kernel(st_ref, out_ref, sc_ref, *, nbins, nb, rows):
    for chunk in range(0, nbins, _BIN_CHUNK):

        accs = [
            jnp.zeros((rows, out_ref.shape[-1]), jnp.int16)
            for _ in range(_BIN_CHUNK)
        ]
        for b in range(nb):
            if chunk == 0:
                tile = st_ref[b, 0, :, :].astype(jnp.int16)
                sc_ref[b, :, :] = tile
            else:
                tile = sc_ref[b, :, :]
            for i in range(_BIN_CHUNK):
                accs[i] = accs[i] + (
                    tile == jnp.int16(chunk + i)
                ).astype(jnp.int16)
        for i, acc in enumerate(accs):
            out_ref[0, chunk + i, :, :] = (
                (acc[0:8, :] + acc[8:16, :]) + (acc[16:24, :] + acc[24:32, :])
            )


def _band_counts(seg, nV, P):
    B, H, W = seg.shape
    rows = H // P

    st = seg.reshape(B, P, rows, W)

    kernel_body = functools.partial(
        _band_hist_kernel, nbins=nV, nb=B, rows=rows
    )

    return pl.pallas_call(
        kernel_body,
        out_shape=jax.ShapeDtypeStruct((P, nV, 8, W), jnp.int16),
        grid=(P,),
        in_specs=[
            pl.BlockSpec((B, 1, rows, W), lambda hp: (0, hp, 0, 0))
        ],
        out_specs=pl.BlockSpec((1, nV, 8, W), lambda hp: (hp, 0, 0, 0)),
        scratch_shapes=[pltpu.VMEM((B, rows, W), jnp.int16)],
        compiler_params=pltpu.CompilerParams(
            dimension_semantics=("parallel",)
        ),
    )(st)


def kernel(seg, byx):
    del byx
    nV, P = _NV, _P
    pps = P
    B, H, W = seg.shape
    ws = W // P

    partial = _band_counts(seg.astype(jnp.int32), nV, P)

    counts = partial.reshape(P, nV, 8, P, ws).sum(
        axis=(2, 4), dtype=jnp.int32
    )
    grid = (
        counts.transpose(1, 0, 2)
        .astype(jnp.float32)
        .reshape(nV, 1, P, P)
    )
    sizes = counts.sum(axis=(0, 2)).astype(jnp.float32)
    den = sizes * (pps / 32.0) ** 2
    return grid / den.reshape(-1, 1, 1, 1)


# speedup vs baseline: 1.7478x; 1.4446x over previous
import functools

import jax
import jax.numpy as jnp
from jax.experimental import pallas as pl
from jax.experimental.pallas import tpu as pltpu


_NV = 64
_P = 8
_PAIR_CHUNK = 4


def _band_hist_kernel(st_ref, out_ref, th_ref, pt_ref, *, nbins, nb, rows):
    for chunk in range(0, nbins // 2, _PAIR_CHUNK):

        accs = [
            jnp.zeros((rows, out_ref.shape[-1]), jnp.int16)
            for _ in range(_PAIR_CHUNK)
        ]
        for b in range(nb):
            if chunk == 0:
                t32 = st_ref[b, 0, :, :]
                th = (t32 >> 1).astype(jnp.int16)
                pt = jnp.where((t32 & 1) != 0, 64, 1).astype(jnp.int16)
                th_ref[b, :, :] = th
                pt_ref[b, :, :] = pt
            else:
                th = th_ref[b, :, :]
                pt = pt_ref[b, :, :]
            for i in range(_PAIR_CHUNK):
                accs[i] = accs[i] + jnp.where(
                    th == jnp.int16(chunk + i), pt, jnp.int16(0)
                )
        for i, acc in enumerate(accs):
            even = acc & jnp.int16(63)
            odd64 = acc - even
            out_ref[0, 2 * (chunk + i), :, :] = (
                (even[0:8, :] + even[8:16, :]) + (even[16:24, :] + even[24:32, :])
            )
            out_ref[0, 2 * (chunk + i) + 1, :, :] = (
                (odd64[0:8, :] + odd64[8:16, :]) + (odd64[16:24, :] + odd64[24:32, :])
            )


def _band_counts(seg, nV, P):
    B, H, W = seg.shape
    rows = H // P

    st = seg.reshape(B, P, rows, W)

    kernel_body = functools.partial(
        _band_hist_kernel, nbins=nV, nb=B, rows=rows
    )

    return pl.pallas_call(
        kernel_body,
        out_shape=jax.ShapeDtypeStruct((P, nV, 8, W), jnp.int16),
        grid=(P,),
        in_specs=[
            pl.BlockSpec((B, 1, rows, W), lambda hp: (0, hp, 0, 0))
        ],
        out_specs=pl.BlockSpec((1, nV, 8, W), lambda hp: (hp, 0, 0, 0)),
        scratch_shapes=[pltpu.VMEM((B, rows, W), jnp.int16),
                        pltpu.VMEM((B, rows, W), jnp.int16)],
        compiler_params=pltpu.CompilerParams(
            dimension_semantics=("parallel",)
        ),
    )(st)


def kernel(seg, byx):
    del byx
    nV, P = _NV, _P
    pps = P
    B, H, W = seg.shape
    ws = W // P

    partial = _band_counts(seg.astype(jnp.int32), nV, P)

    counts = partial.reshape(P, nV, 8, P, ws).sum(
        axis=(2, 4), dtype=jnp.int32
    )
    counts = counts // jnp.where(
        (jnp.arange(nV) & 1) == 1, 64, 1
    ).reshape(1, nV, 1)
    grid = (
        counts.transpose(1, 0, 2)
        .astype(jnp.float32)
        .reshape(nV, 1, P, P)
    )
    sizes = counts.sum(axis=(0, 2)).astype(jnp.float32)
    den = sizes * (pps / 32.0) ** 2
    return grid / den.reshape(-1, 1, 1, 1)
